# baseline (speedup 1.0000x reference)
import functools

import jax
import jax.numpy as jnp
from jax import lax
from jax.experimental import pallas as pl
from jax.experimental.pallas import tpu as pltpu
from jax.experimental.pallas import tpu_sc as plsc

_NC = 2
_NS = 16
_NW = _NC * _NS

_B = 1024
_P = 1025
_D = 64
_ROWS = 32
_NQ = 4
_CHD = _D // _NQ


def _add_splat_chunk(buf, tvec):
    for dd in range(_CHD):
        splat = jnp.full((16,), tvec[dd], jnp.float32)

        @plsc.parallel_loop(0, _B, step=16, unroll=8)
        def _bb(off, dd=dd, splat=splat):
            plsc.addupdate(buf.at[dd, pl.ds(off, 16)], splat)


def _sc_body(inp_hbm, tab_hbm, out_hbm, tab_v, tabt_v, buf0, buf1, buf2,
             buf3, sem0, sem1, sem2, sem3):
    w = lax.axis_index("s") * _NC + lax.axis_index("c")
    p0 = w * _ROWS
    bufs = (buf0, buf1, buf2, buf3)
    sems = (sem0, sem1, sem2, sem3)

    pltpu.sync_copy(tab_hbm.at[pl.ds(p0 * _D, (_ROWS + 1) * _D)], tab_v)
    pltpu.sync_copy(tab_hbm.at[pl.ds(_P * _D - _D, _D)], tabt_v)

    def fire(r, q):
        dst = out_hbm.at[p0 + r, pl.ds(q * _CHD, _CHD), :]
        pltpu.async_copy(bufs[q], dst, sems[q])

    def wait(r, q):
        dst = out_hbm.at[p0 + r, pl.ds(q * _CHD, _CHD), :]
        pltpu.make_async_copy(bufs[q], dst, sems[q]).wait()

    for q in range(_NQ):
        pltpu.sync_copy(inp_hbm.at[pl.ds(q * _CHD, _CHD), :], bufs[q])
        _add_splat_chunk(bufs[q], tab_v[pl.ds(q * _CHD, _CHD)])
        fire(0, q)

    def row_body(r, _):
        for q in range(_NQ):
            wait(r - 1, q)
            tvec = (tab_v[pl.ds(r * _D + q * _CHD, _CHD)]
                    - tab_v[pl.ds((r - 1) * _D + q * _CHD, _CHD)])
            _add_splat_chunk(bufs[q], tvec)
            fire(r, q)
        return 0

    lax.fori_loop(1, _ROWS, row_body, 0)

    @pl.when(w == 0)
    def _():
        for q in range(_NQ):
            wait(_ROWS - 1, q)
            tvec = (tabt_v[pl.ds(q * _CHD, _CHD)]
                    - tab_v[pl.ds((_ROWS - 1) * _D + q * _CHD, _CHD)])
            _add_splat_chunk(bufs[q], tvec)
            dst = out_hbm.at[_P - 1, pl.ds(q * _CHD, _CHD), :]
            pltpu.async_copy(bufs[q], dst, sems[q])
            pltpu.make_async_copy(bufs[q], dst, sems[q]).wait()

    @pl.when(w != 0)
    def _():
        for q in range(_NQ):
            wait(_ROWS - 1, q)


def sc_kernel(inputs, table):
    B = inputs.shape[0]
    P, D = table.shape
    inp_t = inputs.reshape(B, D).T
    tab_flat = table.reshape(P * D)
    mesh = plsc.VectorSubcoreMesh(core_axis_name="c", subcore_axis_name="s")
    run = functools.partial(
        pl.kernel,
        mesh=mesh,
        out_type=jax.ShapeDtypeStruct((P, D, B), jnp.float32),
        scratch_types=[
            pltpu.VMEM(((_ROWS + 1) * D,), jnp.float32),
            pltpu.VMEM((D,), jnp.float32),
            pltpu.VMEM((_CHD, B), jnp.float32),
            pltpu.VMEM((_CHD, B), jnp.float32),
            pltpu.VMEM((_CHD, B), jnp.float32),
            pltpu.VMEM((_CHD, B), jnp.float32),
            pltpu.SemaphoreType.DMA,
            pltpu.SemaphoreType.DMA,
            pltpu.SemaphoreType.DMA,
            pltpu.SemaphoreType.DMA,
        ],
    )(_sc_body)
    phys = run(inp_t, tab_flat)
    return phys.transpose(2, 0, 1)


kernel = sc_kernel
